# baseline (speedup 1.0000x reference)
import functools

import jax
import jax.numpy as jnp
from jax import lax
from jax.experimental import pallas as pl
from jax.experimental.pallas import tpu as pltpu
from jax.experimental.pallas import tpu_sc as plsc

NC = 2
NS = 16
L = 16
NW = NC * NS

B = 16384
D = 128
BPW = B // NW
CHUNK = 128
NCHUNK = BPW // CHUNK
GROUPS = CHUNK // L
KD = D // L

_mesh = plsc.VectorSubcoreMesh(
    core_axis_name="c", subcore_axis_name="s", num_cores=NC, num_subcores=NS
)


@functools.partial(
    pl.kernel,
    out_type=jax.ShapeDtypeStruct((B,), jnp.float32),
    mesh=_mesh,
    compiler_params=pltpu.CompilerParams(needs_layout_passes=False),
    scratch_types=[
        pltpu.VMEM((CHUNK,), jnp.int32),
        pltpu.VMEM((CHUNK,), jnp.int32),
        pltpu.VMEM((CHUNK, D), jnp.float32),
        pltpu.VMEM((CHUNK, D), jnp.float32),
        pltpu.VMEM((D,), jnp.float32),
        pltpu.VMEM((L,), jnp.float32),
        pltpu.VMEM((BPW,), jnp.float32),
        pltpu.SemaphoreType.DMA,
    ],
)
def _mf_kernel(uid_hbm, mid_hbm, ut_hbm, mt_hbm, w_hbm, b_hbm, out_hbm,
               uidx_v, midx_v, urows_v, mrows_v, w_v, b_v, out_v, sem):
    wid = lax.axis_index("s") * NC + lax.axis_index("c")
    base = wid * BPW

    pltpu.sync_copy(w_hbm, w_v)
    pltpu.sync_copy(b_hbm, b_v)
    wk = [w_v[pl.ds(k * L, L)] for k in range(KD)]
    bvec = b_v[...]
    lane = lax.broadcasted_iota(jnp.int32, (L,), 0)
    perms = [jnp.bitwise_xor(lane, 1 << s) for s in range(4)]

    def chunk_body(c, carry):
        off = base + c * CHUNK
        pltpu.sync_copy(uid_hbm.at[pl.ds(off, CHUNK)], uidx_v)
        pltpu.sync_copy(mid_hbm.at[pl.ds(off, CHUNK)], midx_v)
        cu = pltpu.async_copy(ut_hbm.at[uidx_v], urows_v, sem)
        cm = pltpu.async_copy(mt_hbm.at[midx_v], mrows_v, sem)
        cu.wait()
        cm.wait()

        def group_body(g, gcarry):
            res = bvec
            for j in range(L):
                r = g * L + j
                acc = urows_v[r, pl.ds(0, L)] * mrows_v[r, pl.ds(0, L)] * wk[0]
                for k in range(1, KD):
                    acc = acc + (urows_v[r, pl.ds(k * L, L)]
                                 * mrows_v[r, pl.ds(k * L, L)] * wk[k])
                s = jnp.sum(acc)
                res = jnp.where(lane == j, s, res)
            out_v[pl.ds(c * CHUNK + g * L, L)] = res + bvec
            return gcarry

        lax.fori_loop(0, GROUPS, group_body, 0)
        return carry

    lax.fori_loop(0, NCHUNK, chunk_body, 0)
    pltpu.sync_copy(out_v, out_hbm.at[pl.ds(base, BPW)])


def kernel(user_ids, movie_ids, user_table, movie_table, fc_w, fc_b):
    uid = user_ids.astype(jnp.int32)
    mid = movie_ids.astype(jnp.int32)
    w = fc_w.reshape(D).astype(jnp.float32)
    b16 = jnp.broadcast_to(fc_b.astype(jnp.float32), (L,))
    return _mf_kernel(uid, mid, user_table, movie_table, w, b16)


# speedup vs baseline: 1.0116x; 1.0116x over previous
import functools

import jax
import jax.numpy as jnp
from jax import lax
from jax.experimental import pallas as pl
from jax.experimental.pallas import tpu as pltpu
from jax.experimental.pallas import tpu_sc as plsc

NC = 2
NS = 16
L = 16
NW = NC * NS

B = 16384
D = 128
BPW = B // NW
CHUNK = 128
NCHUNK = BPW // CHUNK
GROUPS = CHUNK // L
KD = D // L

_mesh = plsc.VectorSubcoreMesh(
    core_axis_name="c", subcore_axis_name="s", num_cores=NC, num_subcores=NS
)


@functools.partial(
    pl.kernel,
    out_type=jax.ShapeDtypeStruct((B,), jnp.float32),
    mesh=_mesh,
    compiler_params=pltpu.CompilerParams(needs_layout_passes=False),
    scratch_types=[
        pltpu.VMEM((BPW,), jnp.int32),
        pltpu.VMEM((BPW,), jnp.int32),
        pltpu.VMEM((CHUNK, D), jnp.float32),
        pltpu.VMEM((CHUNK, D), jnp.float32),
        pltpu.VMEM((CHUNK, D), jnp.float32),
        pltpu.VMEM((CHUNK, D), jnp.float32),
        pltpu.VMEM((D,), jnp.float32),
        pltpu.VMEM((L,), jnp.float32),
        pltpu.VMEM((BPW,), jnp.float32),
        pltpu.SemaphoreType.DMA,
        pltpu.SemaphoreType.DMA,
        pltpu.SemaphoreType.DMA,
        pltpu.SemaphoreType.DMA,
    ],
)
def _mf_kernel(uid_hbm, mid_hbm, ut_hbm, mt_hbm, w_hbm, b_hbm, out_hbm,
               uidx_v, midx_v, urows0, urows1, mrows0, mrows1,
               w_v, b_v, out_v, semu0, semu1, semm0, semm1):
    wid = lax.axis_index("s") * NC + lax.axis_index("c")
    base = wid * BPW

    pltpu.sync_copy(uid_hbm.at[pl.ds(base, BPW)], uidx_v)
    pltpu.sync_copy(mid_hbm.at[pl.ds(base, BPW)], midx_v)
    pltpu.sync_copy(w_hbm, w_v)
    pltpu.sync_copy(b_hbm, b_v)
    wk = [w_v[pl.ds(k * L, L)] for k in range(KD)]
    bvec = b_v[...]
    lane = lax.broadcasted_iota(jnp.int32, (L,), 0)

    urows = (urows0, urows1)
    mrows = (mrows0, mrows1)
    semu = (semu0, semu1)
    semm = (semm0, semm1)

    def start(c):
        buf = c % 2
        cu = pltpu.async_copy(
            ut_hbm.at[uidx_v.at[pl.ds(c * CHUNK, CHUNK)]], urows[buf], semu[buf])
        cm = pltpu.async_copy(
            mt_hbm.at[midx_v.at[pl.ds(c * CHUNK, CHUNK)]], mrows[buf], semm[buf])
        return cu, cm

    inflight = {0: start(0), 1: start(1)}

    for c in range(NCHUNK):
        buf = c % 2
        cu, cm = inflight.pop(c)
        cu.wait()
        cm.wait()
        u_v, m_v = urows[buf], mrows[buf]

        def group_body(g, gcarry, c=c, u_v=u_v, m_v=m_v):
            res = bvec
            for j in range(L):
                r = g * L + j
                acc = u_v[r, pl.ds(0, L)] * m_v[r, pl.ds(0, L)] * wk[0]
                for k in range(1, KD):
                    acc = acc + (u_v[r, pl.ds(k * L, L)]
                                 * m_v[r, pl.ds(k * L, L)] * wk[k])
                s = jnp.sum(acc)
                res = jnp.where(lane == j, s, res)
            out_v[pl.ds(c * CHUNK + g * L, L)] = res + bvec
            return gcarry

        lax.fori_loop(0, GROUPS, group_body, 0)
        if c + 2 < NCHUNK:
            inflight[c + 2] = start(c + 2)

    pltpu.sync_copy(out_v, out_hbm.at[pl.ds(base, BPW)])


def kernel(user_ids, movie_ids, user_table, movie_table, fc_w, fc_b):
    uid = user_ids.astype(jnp.int32)
    mid = movie_ids.astype(jnp.int32)
    w = fc_w.reshape(D).astype(jnp.float32)
    b16 = jnp.broadcast_to(fc_b.astype(jnp.float32), (L,))
    return _mf_kernel(uid, mid, user_table, movie_table, w, b16)


# speedup vs baseline: 1.2124x; 1.1985x over previous
import functools

import jax
import jax.numpy as jnp
from jax import lax
from jax.experimental import pallas as pl
from jax.experimental.pallas import tpu as pltpu
from jax.experimental.pallas import tpu_sc as plsc

NC = 2
NS = 16
L = 16
NW = NC * NS

B = 16384
D = 128
BPW = B // NW
CHUNK = 128
NCHUNK = BPW // CHUNK
GROUPS = CHUNK // L
KD = D // L

_mesh = plsc.VectorSubcoreMesh(
    core_axis_name="c", subcore_axis_name="s", num_cores=NC, num_subcores=NS
)


@functools.partial(
    pl.kernel,
    out_type=jax.ShapeDtypeStruct((B,), jnp.float32),
    mesh=_mesh,
    compiler_params=pltpu.CompilerParams(needs_layout_passes=False),
    scratch_types=[
        pltpu.VMEM((BPW,), jnp.int32),
        pltpu.VMEM((BPW,), jnp.int32),
        pltpu.VMEM((CHUNK, D), jnp.float32),
        pltpu.VMEM((CHUNK, D), jnp.float32),
        pltpu.VMEM((CHUNK, D), jnp.float32),
        pltpu.VMEM((CHUNK, D), jnp.float32),
        pltpu.VMEM((D,), jnp.float32),
        pltpu.VMEM((L,), jnp.float32),
        pltpu.VMEM((BPW,), jnp.float32),
        pltpu.VMEM((L, L), jnp.float32),
        pltpu.SemaphoreType.DMA,
        pltpu.SemaphoreType.DMA,
        pltpu.SemaphoreType.DMA,
        pltpu.SemaphoreType.DMA,
    ],
)
def _mf_kernel(uid_hbm, mid_hbm, ut_hbm, mt_hbm, w_hbm, b_hbm, out_hbm,
               uidx_v, midx_v, urows0, urows1, mrows0, mrows1,
               w_v, b_v, out_v, acc_v, semu0, semu1, semm0, semm1):
    wid = lax.axis_index("s") * NC + lax.axis_index("c")
    base = wid * BPW

    pltpu.sync_copy(uid_hbm.at[pl.ds(base, BPW)], uidx_v)
    pltpu.sync_copy(mid_hbm.at[pl.ds(base, BPW)], midx_v)
    pltpu.sync_copy(w_hbm, w_v)
    pltpu.sync_copy(b_hbm, b_v)
    wk = [w_v[pl.ds(k * L, L)] for k in range(KD)]
    bvec = b_v[...]
    lane = lax.broadcasted_iota(jnp.int32, (L,), 0)

    urows = (urows0, urows1)
    mrows = (mrows0, mrows1)
    semu = (semu0, semu1)
    semm = (semm0, semm1)

    def start(c):
        buf = c % 2
        cu = pltpu.async_copy(
            ut_hbm.at[uidx_v.at[pl.ds(c * CHUNK, CHUNK)]], urows[buf], semu[buf])
        cm = pltpu.async_copy(
            mt_hbm.at[midx_v.at[pl.ds(c * CHUNK, CHUNK)]], mrows[buf], semm[buf])
        return cu, cm

    inflight = {0: start(0), 1: start(1)}

    for c in range(NCHUNK):
        buf = c % 2
        cu, cm = inflight.pop(c)
        cu.wait()
        cm.wait()
        u_v, m_v = urows[buf], mrows[buf]

        def group_body(g, gcarry, c=c, u_v=u_v, m_v=m_v):
            row0 = g * L
            for j in range(L):
                r = row0 + j
                acc0 = u_v[r, pl.ds(0, L)] * m_v[r, pl.ds(0, L)] * wk[0]
                acc1 = u_v[r, pl.ds(L, L)] * m_v[r, pl.ds(L, L)] * wk[1]
                for k in range(2, KD, 2):
                    acc0 = acc0 + (u_v[r, pl.ds(k * L, L)]
                                   * m_v[r, pl.ds(k * L, L)] * wk[k])
                    acc1 = acc1 + (u_v[r, pl.ds((k + 1) * L, L)]
                                   * m_v[r, pl.ds((k + 1) * L, L)] * wk[k + 1])
                acc_v[j, :] = acc0 + acc1
            cols = [plsc.load_gather(acc_v, [lane, jnp.full((L,), l, jnp.int32)])
                    for l in range(L)]
            while len(cols) > 1:
                cols = [cols[i] + cols[i + 1] for i in range(0, len(cols), 2)]
            out_v[pl.ds(c * CHUNK + row0, L)] = cols[0] + bvec
            return gcarry

        lax.fori_loop(0, GROUPS, group_body, 0)
        if c + 2 < NCHUNK:
            inflight[c + 2] = start(c + 2)

    pltpu.sync_copy(out_v, out_hbm.at[pl.ds(base, BPW)])


def kernel(user_ids, movie_ids, user_table, movie_table, fc_w, fc_b):
    uid = user_ids.astype(jnp.int32)
    mid = movie_ids.astype(jnp.int32)
    w = fc_w.reshape(D).astype(jnp.float32)
    b16 = jnp.broadcast_to(fc_b.astype(jnp.float32), (L,))
    return _mf_kernel(uid, mid, user_table, movie_table, w, b16)
